# baseline (speedup 1.0000x reference)
import functools

import jax
import jax.numpy as jnp
from jax import lax
from jax.experimental import pallas as pl
from jax.experimental.pallas import tpu as pltpu
from jax.experimental.pallas import tpu_sc as plsc

_NC = 2
_NS = 16
_NW = _NC * _NS
_GCHUNK = 128
_ZCHUNK = 80

N_EDGE_BLK = 1280
N_NODE_BLK = 1000
D_X = 256
D_E = 16
D_H = 544
D_U = 272
D_SLAB = 128
D_TAIL = 128


def _sc_gather(x, idx2d):
    n_chunks = idx2d.shape[0]
    d = x.shape[1]
    per_w = (n_chunks + _NW - 1) // _NW
    mesh = plsc.VectorSubcoreMesh(core_axis_name="c", subcore_axis_name="s")

    @functools.partial(
        pl.kernel,
        mesh=mesh,
        out_type=jax.ShapeDtypeStruct((n_chunks * _GCHUNK, d), jnp.float32),
        scratch_types=[
            pltpu.VMEM((_GCHUNK,), jnp.int32),
            pltpu.VMEM((_GCHUNK,), jnp.int32),
            pltpu.VMEM((_GCHUNK, d), jnp.float32),
            pltpu.VMEM((_GCHUNK, d), jnp.float32),
            pltpu.SemaphoreType.DMA,
            pltpu.SemaphoreType.DMA,
            pltpu.SemaphoreType.DMA,
            pltpu.SemaphoreType.DMA,
            pltpu.SemaphoreType.DMA,
            pltpu.SemaphoreType.DMA,
        ],
    )
    def k(x_hbm, idx_hbm, out_hbm, idx0, idx1, rows0, rows1, is0, is1, gs0, gs1, ws0, ws1):
        wid = lax.axis_index("s") * _NC + lax.axis_index("c")
        idxs = (idx0, idx1)
        rows = (rows0, rows1)
        isems = (is0, is1)
        gsems = (gs0, gs1)
        wsems = (ws0, ws1)

        def cid_of(i):
            return wid + _NW * i

        @pl.when(cid_of(0) < n_chunks)
        def _():
            pltpu.async_copy(idx_hbm.at[cid_of(0)], idx0, is0)

        def half(i, b):
            cid = cid_of(i)
            cidm1 = cid_of(i - 1)
            cidm2 = cid_of(i - 2)
            bo = 1 - b

            @pl.when(jnp.logical_and(i >= 2, cidm2 < n_chunks))
            def _():
                pltpu.make_async_copy(
                    rows[b], out_hbm.at[pl.ds(cidm2 * _GCHUNK, _GCHUNK)], wsems[b]
                ).wait()

            @pl.when(cid < n_chunks)
            def _():
                pltpu.make_async_copy(idx_hbm.at[cid], idxs[b], isems[b]).wait()
                pltpu.async_copy(x_hbm.at[idxs[b]], rows[b], gsems[b])

            @pl.when(jnp.logical_and(i >= 1, cidm1 < n_chunks))
            def _():
                pltpu.make_async_copy(x_hbm.at[idxs[bo]], rows[bo], gsems[bo]).wait()
                pltpu.async_copy(
                    rows[bo], out_hbm.at[pl.ds(cidm1 * _GCHUNK, _GCHUNK)], wsems[bo]
                )

            @pl.when(cid_of(i + 1) < n_chunks)
            def _():
                pltpu.async_copy(idx_hbm.at[cid_of(i + 1)], idxs[bo], isems[bo])

        def body(o, carry):
            half(2 * o, 0)
            half(2 * o + 1, 1)
            return carry

        lax.fori_loop(0, (per_w + 2 + 1) // 2 + 1, body, 0)

    return k(x, idx2d)


def _zero_acc(data_v, acc, sid, n_zchunks, width):
    def zrow(i, c):
        for j in range(width // 16):
            data_v[i, pl.ds(j * 16, 16)] = jnp.zeros((16,), jnp.float32)
        return c

    lax.fori_loop(0, _ZCHUNK, zrow, 0)

    def zblk(i, c):
        k = sid + _NS * i

        @pl.when(k < n_zchunks)
        def _():
            pltpu.sync_copy(data_v.at[pl.ds(0, _ZCHUNK)], acc.at[pl.ds(k * _ZCHUNK, _ZCHUNK)])

        return c

    lax.fori_loop(0, (n_zchunks + _NS - 1) // _NS, zblk, 0)


def _sc_scatter_main(u01, col2d, n_nodes):
    n_chunks = col2d.shape[0]
    n_zchunks = n_nodes // _ZCHUNK
    mesh = plsc.VectorSubcoreMesh(core_axis_name="c", subcore_axis_name="s")

    @functools.partial(
        pl.kernel,
        mesh=mesh,
        out_type=jax.ShapeDtypeStruct((n_nodes, 2 * D_SLAB), jnp.float32),
        scratch_types=[
            pltpu.VMEM((_GCHUNK,), jnp.int32),
            pltpu.VMEM((_GCHUNK,), jnp.int32),
            pltpu.VMEM((_GCHUNK, D_SLAB), jnp.float32),
            pltpu.VMEM((_GCHUNK, D_SLAB), jnp.float32),
            pltpu.VMEM_SHARED((n_nodes, D_SLAB), jnp.float32),
            pltpu.SemaphoreType.DMA,
            pltpu.SemaphoreType.DMA,
            pltpu.SemaphoreType.DMA,
            pltpu.SemaphoreType.DMA,
        ],
    )
    def k(u_hbm, col_hbm, s_hbm, idx0, idx1, data0, data1, acc, is0, is1, ds0, ds1):
        core = lax.axis_index("c")
        sid = lax.axis_index("s")
        idxs = (idx0, idx1)
        datas = (data0, data1)
        isems = (is0, is1)
        dsems = (ds0, ds1)

        _zero_acc(data0, acc, sid, n_zchunks, D_SLAB)
        plsc.subcore_barrier()

        def cid_of(i):
            return sid + _NS * i

        def u_at(cid):
            return u_hbm.at[core, pl.ds(cid * _GCHUNK, _GCHUNK)]

        @pl.when(cid_of(0) < n_chunks)
        def _():
            pltpu.async_copy(col_hbm.at[cid_of(0)], idx0, is0)
            pltpu.async_copy(u_at(cid_of(0)), data0, ds0)

        def half(i, b):
            bo = 1 - b

            @pl.when(cid_of(i + 1) < n_chunks)
            def _():
                pltpu.async_copy(col_hbm.at[cid_of(i + 1)], idxs[bo], isems[bo])
                pltpu.async_copy(u_at(cid_of(i + 1)), datas[bo], dsems[bo])

            @pl.when(cid_of(i) < n_chunks)
            def _():
                pltpu.make_async_copy(col_hbm.at[cid_of(i)], idxs[b], isems[b]).wait()
                pltpu.make_async_copy(u_at(cid_of(i)), datas[b], dsems[b]).wait()
                pltpu.sync_copy(datas[b], acc.at[idxs[b]], add=True)

        def body(o, carry):
            half(2 * o, 0)
            half(2 * o + 1, 1)
            return carry

        per_tile = (n_chunks + _NS - 1) // _NS
        lax.fori_loop(0, (per_tile + 1) // 2 + 1, body, 0)
        plsc.subcore_barrier()

        def wblk(i, c):
            k = sid + _NS * i

            @pl.when(k < n_zchunks)
            def _():
                r0 = k * _ZCHUNK
                pltpu.sync_copy(acc.at[pl.ds(r0, _ZCHUNK)], data0.at[pl.ds(0, _ZCHUNK)])
                pltpu.sync_copy(
                    data0.at[pl.ds(0, _ZCHUNK)],
                    s_hbm.at[pl.ds(r0, _ZCHUNK), pl.ds(core * D_SLAB, D_SLAB)],
                )

            return c

        lax.fori_loop(0, (n_zchunks + _NS - 1) // _NS, wblk, 0)

    return k(u01, col2d)


def _sc_scatter_tail(u2, col2d, n_nodes):
    n_chunks = col2d.shape[0]
    per_core = n_chunks // _NC
    n_zchunks = n_nodes // _ZCHUNK
    mesh = plsc.VectorSubcoreMesh(core_axis_name="c", subcore_axis_name="s")

    @functools.partial(
        pl.kernel,
        mesh=mesh,
        out_type=jax.ShapeDtypeStruct((_NC, n_nodes, D_TAIL), jnp.float32),
        scratch_types=[
            pltpu.VMEM((_GCHUNK,), jnp.int32),
            pltpu.VMEM((_GCHUNK,), jnp.int32),
            pltpu.VMEM((_GCHUNK, D_TAIL), jnp.float32),
            pltpu.VMEM((_GCHUNK, D_TAIL), jnp.float32),
            pltpu.VMEM_SHARED((n_nodes, D_TAIL), jnp.float32),
            pltpu.SemaphoreType.DMA,
            pltpu.SemaphoreType.DMA,
            pltpu.SemaphoreType.DMA,
            pltpu.SemaphoreType.DMA,
        ],
    )
    def k(u_hbm, col_hbm, s_hbm, idx0, idx1, data0, data1, acc, is0, is1, ds0, ds1):
        core = lax.axis_index("c")
        sid = lax.axis_index("s")
        idxs = (idx0, idx1)
        datas = (data0, data1)
        isems = (is0, is1)
        dsems = (ds0, ds1)

        _zero_acc(data0, acc, sid, n_zchunks, D_TAIL)
        plsc.subcore_barrier()

        def j_of(i):
            return sid + _NS * i

        def cid_of(i):
            return core + _NC * j_of(i)

        def u_at(cid):
            return u_hbm.at[pl.ds(cid * _GCHUNK, _GCHUNK)]

        @pl.when(j_of(0) < per_core)
        def _():
            pltpu.async_copy(col_hbm.at[cid_of(0)], idx0, is0)
            pltpu.async_copy(u_at(cid_of(0)), data0, ds0)

        def half(i, b):
            bo = 1 - b

            @pl.when(j_of(i + 1) < per_core)
            def _():
                pltpu.async_copy(col_hbm.at[cid_of(i + 1)], idxs[bo], isems[bo])
                pltpu.async_copy(u_at(cid_of(i + 1)), datas[bo], dsems[bo])

            @pl.when(j_of(i) < per_core)
            def _():
                pltpu.make_async_copy(col_hbm.at[cid_of(i)], idxs[b], isems[b]).wait()
                pltpu.make_async_copy(u_at(cid_of(i)), datas[b], dsems[b]).wait()
                pltpu.sync_copy(datas[b], acc.at[idxs[b]], add=True)

        def body(o, carry):
            half(2 * o, 0)
            half(2 * o + 1, 1)
            return carry

        per_tile = (per_core + _NS - 1) // _NS
        lax.fori_loop(0, (per_tile + 1) // 2 + 1, body, 0)
        plsc.subcore_barrier()

        def wblk(i, c):
            k = sid + _NS * i

            @pl.when(k < n_zchunks)
            def _():
                r0 = k * _ZCHUNK
                pltpu.sync_copy(acc.at[pl.ds(r0, _ZCHUNK)], data0.at[pl.ds(0, _ZCHUNK)])
                pltpu.sync_copy(
                    data0.at[pl.ds(0, _ZCHUNK)], s_hbm.at[core, pl.ds(r0, _ZCHUNK)]
                )

            return c

        lax.fori_loop(0, (n_zchunks + _NS - 1) // _NS, wblk, 0)

    return k(u2, col2d)


def _edge_block(
    xg_ref, ea_ref, w1x_ref, w1e_ref, b1_ref, wc0_ref, wc1_ref, wc2_ref, cvec_ref,
    o01_ref, o2_ref,
):
    xb = xg_ref[...].astype(jnp.bfloat16)
    t = jnp.dot(xb, w1x_ref[...], preferred_element_type=jnp.float32)
    t += jnp.dot(
        ea_ref[...].astype(jnp.bfloat16), w1e_ref[...], preferred_element_type=jnp.float32
    )
    t = jnp.maximum(t + b1_ref[...], 0.0).astype(jnp.bfloat16)
    o01_ref[0] = jnp.dot(t, wc0_ref[...], preferred_element_type=jnp.float32)
    o01_ref[1] = jnp.dot(t, wc1_ref[...], preferred_element_type=jnp.float32)
    o2_ref[...] = (
        jnp.dot(t, wc2_ref[...], preferred_element_type=jnp.float32) + cvec_ref[...]
    )


def _edge_stage(xg, ea, W1a_x, W1a_e, b1a, Wc0, Wc1, Wc2p, cvec2):
    n_edges = xg.shape[0]
    grid = (n_edges // N_EDGE_BLK,)
    return pl.pallas_call(
        _edge_block,
        grid=grid,
        in_specs=[
            pl.BlockSpec((N_EDGE_BLK, D_X), lambda i: (i, 0)),
            pl.BlockSpec((N_EDGE_BLK, D_E), lambda i: (i, 0)),
            pl.BlockSpec((D_X, D_H), lambda i: (0, 0)),
            pl.BlockSpec((D_E, D_H), lambda i: (0, 0)),
            pl.BlockSpec((D_H,), lambda i: (0,)),
            pl.BlockSpec((D_H, D_SLAB), lambda i: (0, 0)),
            pl.BlockSpec((D_H, D_SLAB), lambda i: (0, 0)),
            pl.BlockSpec((D_H, D_TAIL), lambda i: (0, 0)),
            pl.BlockSpec((D_TAIL,), lambda i: (0,)),
        ],
        out_specs=[
            pl.BlockSpec((2, N_EDGE_BLK, D_SLAB), lambda i: (0, i, 0)),
            pl.BlockSpec((N_EDGE_BLK, D_TAIL), lambda i: (i, 0)),
        ],
        out_shape=[
            jax.ShapeDtypeStruct((2, n_edges, D_SLAB), jnp.float32),
            jax.ShapeDtypeStruct((n_edges, D_TAIL), jnp.float32),
        ],
    )(xg, ea, W1a_x, W1a_e, b1a, Wc0, Wc1, Wc2p, cvec2)


def _node_block(x_ref, s01_ref, s2_ref, w2x_ref, b2a_ref, bc_ref, w2b_ref, b2b_ref, o_ref):
    s2 = s2_ref[0] + s2_ref[1]
    cnt = s2[:, 16:17]
    cntc = jnp.maximum(cnt, 1.0)
    s_u = jnp.concatenate([s01_ref[...], s2[:, :16]], axis=1)
    mean_u = s_u / cntc + jnp.where(cnt > 0.0, bc_ref[...][None, :], 0.0)
    h = jnp.dot(x_ref[...], w2x_ref[...], preferred_element_type=jnp.float32)
    h = jnp.maximum(h + mean_u + b2a_ref[...], 0.0)
    o_ref[...] = (
        jnp.dot(h, w2b_ref[...], preferred_element_type=jnp.float32) + b2b_ref[...]
    )


def _node_stage(x, S01, S2, W2a_x, b2a, bc, W2b, b2b):
    n_nodes = x.shape[0]
    grid = (n_nodes // N_NODE_BLK,)
    return pl.pallas_call(
        _node_block,
        grid=grid,
        in_specs=[
            pl.BlockSpec((N_NODE_BLK, D_X), lambda i: (i, 0)),
            pl.BlockSpec((N_NODE_BLK, 2 * D_SLAB), lambda i: (i, 0)),
            pl.BlockSpec((2, N_NODE_BLK, D_TAIL), lambda i: (0, i, 0)),
            pl.BlockSpec((D_X, D_U), lambda i: (0, 0)),
            pl.BlockSpec((D_U,), lambda i: (0,)),
            pl.BlockSpec((D_U,), lambda i: (0,)),
            pl.BlockSpec((D_U, D_X), lambda i: (0, 0)),
            pl.BlockSpec((D_X,), lambda i: (0,)),
        ],
        out_specs=pl.BlockSpec((N_NODE_BLK, D_X), lambda i: (i, 0)),
        out_shape=jax.ShapeDtypeStruct((n_nodes, D_X), jnp.float32),
    )(x, S01, S2, W2a_x, b2a, bc, W2b, b2b)


def kernel(x, edge_index, edge_attr, W1a, b1a, W1b, b1b, W2a, b2a, W2b, b2b):
    n_nodes = x.shape[0]
    row = edge_index[0].astype(jnp.int32)
    col = edge_index[1].astype(jnp.int32)

    W1a_x, W1a_e = W1a[:D_X], W1a[D_X:]
    W2a_x, W2a_m = W2a[:D_X], W2a[D_X:]
    Wc = W1b @ W2a_m
    bc = b1b @ W2a_m
    Wc0 = Wc[:, :D_SLAB].astype(jnp.bfloat16)
    Wc1 = Wc[:, D_SLAB : 2 * D_SLAB].astype(jnp.bfloat16)
    Wc2p = jnp.pad(Wc[:, 2 * D_SLAB :], ((0, 0), (0, D_TAIL - 16))).astype(jnp.bfloat16)
    cvec2 = (jnp.arange(D_TAIL) == 16).astype(jnp.float32)
    W1a_x = W1a_x.astype(jnp.bfloat16)
    W1a_e = W1a_e.astype(jnp.bfloat16)

    xg = _sc_gather(x, row.reshape(-1, _GCHUNK))
    u01, u2 = _edge_stage(xg, edge_attr, W1a_x, W1a_e, b1a, Wc0, Wc1, Wc2p, cvec2)
    col2d = col.reshape(-1, _GCHUNK)
    S01 = _sc_scatter_main(u01, col2d, n_nodes)
    S2 = _sc_scatter_tail(u2, col2d, n_nodes)
    return _node_stage(x, S01, S2, W2a_x, b2a, bc, W2b, b2b)


# speedup vs baseline: 1.1234x; 1.1234x over previous
import functools

import jax
import jax.numpy as jnp
from jax import lax
from jax.experimental import pallas as pl
from jax.experimental.pallas import tpu as pltpu
from jax.experimental.pallas import tpu_sc as plsc

_NC = 2
_NS = 16
_NW = _NC * _NS
_GCHUNK = 128
_ZCHUNK = 80

N_EDGE_BLK = 1600
N_NODE_BLK = 1000
D_X = 256
D_E = 16
D_H = 544
D_U = 272
D_SLAB = 128
D_TAIL = 128


def _sc_gather(x, idx2d):
    n_chunks = idx2d.shape[0]
    d = x.shape[1]
    per_w = (n_chunks + _NW - 1) // _NW
    mesh = plsc.VectorSubcoreMesh(core_axis_name="c", subcore_axis_name="s")

    @functools.partial(
        pl.kernel,
        mesh=mesh,
        out_type=jax.ShapeDtypeStruct((n_chunks * _GCHUNK, d), jnp.float32),
        scratch_types=[
            pltpu.VMEM((_GCHUNK,), jnp.int32),
            pltpu.VMEM((_GCHUNK,), jnp.int32),
            pltpu.VMEM((_GCHUNK, d), jnp.float32),
            pltpu.VMEM((_GCHUNK, d), jnp.float32),
            pltpu.SemaphoreType.DMA,
            pltpu.SemaphoreType.DMA,
            pltpu.SemaphoreType.DMA,
            pltpu.SemaphoreType.DMA,
            pltpu.SemaphoreType.DMA,
            pltpu.SemaphoreType.DMA,
        ],
    )
    def k(x_hbm, idx_hbm, out_hbm, idx0, idx1, rows0, rows1, is0, is1, gs0, gs1, ws0, ws1):
        wid = lax.axis_index("s") * _NC + lax.axis_index("c")
        idxs = (idx0, idx1)
        rows = (rows0, rows1)
        isems = (is0, is1)
        gsems = (gs0, gs1)
        wsems = (ws0, ws1)

        def cid_of(i):
            return wid + _NW * i

        @pl.when(cid_of(0) < n_chunks)
        def _():
            pltpu.async_copy(idx_hbm.at[cid_of(0)], idx0, is0)

        def half(i, b):
            cid = cid_of(i)
            cidm1 = cid_of(i - 1)
            cidm2 = cid_of(i - 2)
            bo = 1 - b

            @pl.when(jnp.logical_and(i >= 2, cidm2 < n_chunks))
            def _():
                pltpu.make_async_copy(
                    rows[b], out_hbm.at[pl.ds(cidm2 * _GCHUNK, _GCHUNK)], wsems[b]
                ).wait()

            @pl.when(cid < n_chunks)
            def _():
                pltpu.make_async_copy(idx_hbm.at[cid], idxs[b], isems[b]).wait()
                pltpu.async_copy(x_hbm.at[idxs[b]], rows[b], gsems[b])

            @pl.when(jnp.logical_and(i >= 1, cidm1 < n_chunks))
            def _():
                pltpu.make_async_copy(x_hbm.at[idxs[bo]], rows[bo], gsems[bo]).wait()
                pltpu.async_copy(
                    rows[bo], out_hbm.at[pl.ds(cidm1 * _GCHUNK, _GCHUNK)], wsems[bo]
                )

            @pl.when(cid_of(i + 1) < n_chunks)
            def _():
                pltpu.async_copy(idx_hbm.at[cid_of(i + 1)], idxs[bo], isems[bo])

        def body(o, carry):
            half(2 * o, 0)
            half(2 * o + 1, 1)
            return carry

        lax.fori_loop(0, (per_w + 2 + 1) // 2 + 1, body, 0)

    return k(x, idx2d)


def _zero_acc(data_v, acc, sid, n_zchunks, width):
    def zrow(i, c):
        for j in range(width // 16):
            data_v[i, pl.ds(j * 16, 16)] = jnp.zeros((16,), jnp.float32)
        return c

    lax.fori_loop(0, _ZCHUNK, zrow, 0)

    def zblk(i, c):
        k = sid + _NS * i

        @pl.when(k < n_zchunks)
        def _():
            pltpu.sync_copy(data_v.at[pl.ds(0, _ZCHUNK)], acc.at[pl.ds(k * _ZCHUNK, _ZCHUNK)])

        return c

    lax.fori_loop(0, (n_zchunks + _NS - 1) // _NS, zblk, 0)


def _sc_scatter_main(u01, col2d, n_nodes):
    n_chunks = col2d.shape[0]
    n_zchunks = n_nodes // _ZCHUNK
    mesh = plsc.VectorSubcoreMesh(core_axis_name="c", subcore_axis_name="s")

    @functools.partial(
        pl.kernel,
        mesh=mesh,
        out_type=jax.ShapeDtypeStruct((n_nodes, 2 * D_SLAB), jnp.float32),
        scratch_types=[
            pltpu.VMEM((_GCHUNK,), jnp.int32),
            pltpu.VMEM((_GCHUNK,), jnp.int32),
            pltpu.VMEM((_GCHUNK, D_SLAB), jnp.float32),
            pltpu.VMEM((_GCHUNK, D_SLAB), jnp.float32),
            pltpu.VMEM_SHARED((n_nodes, D_SLAB), jnp.float32),
            pltpu.SemaphoreType.DMA,
            pltpu.SemaphoreType.DMA,
            pltpu.SemaphoreType.DMA,
            pltpu.SemaphoreType.DMA,
        ],
    )
    def k(u_hbm, col_hbm, s_hbm, idx0, idx1, data0, data1, acc, is0, is1, ds0, ds1):
        core = lax.axis_index("c")
        sid = lax.axis_index("s")
        idxs = (idx0, idx1)
        datas = (data0, data1)
        isems = (is0, is1)
        dsems = (ds0, ds1)

        _zero_acc(data0, acc, sid, n_zchunks, D_SLAB)
        plsc.subcore_barrier()

        def cid_of(i):
            return sid + _NS * i

        def u_at(cid):
            return u_hbm.at[core, pl.ds(cid * _GCHUNK, _GCHUNK)]

        @pl.when(cid_of(0) < n_chunks)
        def _():
            pltpu.async_copy(col_hbm.at[cid_of(0)], idx0, is0)
            pltpu.async_copy(u_at(cid_of(0)), data0, ds0)

        def half(i, b):
            bo = 1 - b

            @pl.when(cid_of(i + 1) < n_chunks)
            def _():
                pltpu.async_copy(col_hbm.at[cid_of(i + 1)], idxs[bo], isems[bo])
                pltpu.async_copy(u_at(cid_of(i + 1)), datas[bo], dsems[bo])

            @pl.when(cid_of(i) < n_chunks)
            def _():
                pltpu.make_async_copy(col_hbm.at[cid_of(i)], idxs[b], isems[b]).wait()
                pltpu.make_async_copy(u_at(cid_of(i)), datas[b], dsems[b]).wait()
                pltpu.sync_copy(datas[b], acc.at[idxs[b]], add=True)

        def body(o, carry):
            half(2 * o, 0)
            half(2 * o + 1, 1)
            return carry

        per_tile = (n_chunks + _NS - 1) // _NS
        lax.fori_loop(0, (per_tile + 1) // 2 + 1, body, 0)
        plsc.subcore_barrier()

        def wblk(i, c):
            k = sid + _NS * i

            @pl.when(k < n_zchunks)
            def _():
                r0 = k * _ZCHUNK
                pltpu.sync_copy(acc.at[pl.ds(r0, _ZCHUNK)], data0.at[pl.ds(0, _ZCHUNK)])
                pltpu.sync_copy(
                    data0.at[pl.ds(0, _ZCHUNK)],
                    s_hbm.at[pl.ds(r0, _ZCHUNK), pl.ds(core * D_SLAB, D_SLAB)],
                )

            return c

        lax.fori_loop(0, (n_zchunks + _NS - 1) // _NS, wblk, 0)

    return k(u01, col2d)


def _sc_scatter_tail(u2, col2d, n_nodes):
    n_chunks = col2d.shape[0]
    per_core_max = (n_chunks + _NC - 1) // _NC
    n_zchunks = n_nodes // _ZCHUNK
    mesh = plsc.VectorSubcoreMesh(core_axis_name="c", subcore_axis_name="s")

    @functools.partial(
        pl.kernel,
        mesh=mesh,
        out_type=jax.ShapeDtypeStruct((_NC, n_nodes, D_TAIL), jnp.float32),
        scratch_types=[
            pltpu.VMEM((_GCHUNK,), jnp.int32),
            pltpu.VMEM((_GCHUNK,), jnp.int32),
            pltpu.VMEM((_GCHUNK, D_TAIL), jnp.float32),
            pltpu.VMEM((_GCHUNK, D_TAIL), jnp.float32),
            pltpu.VMEM_SHARED((n_nodes, D_TAIL), jnp.float32),
            pltpu.SemaphoreType.DMA,
            pltpu.SemaphoreType.DMA,
            pltpu.SemaphoreType.DMA,
            pltpu.SemaphoreType.DMA,
        ],
    )
    def k(u_hbm, col_hbm, s_hbm, idx0, idx1, data0, data1, acc, is0, is1, ds0, ds1):
        core = lax.axis_index("c")
        sid = lax.axis_index("s")
        idxs = (idx0, idx1)
        datas = (data0, data1)
        isems = (is0, is1)
        dsems = (ds0, ds1)

        _zero_acc(data0, acc, sid, n_zchunks, D_TAIL)
        plsc.subcore_barrier()

        def j_of(i):
            return sid + _NS * i

        per_core = (n_chunks - core + _NC - 1) // _NC

        def cid_of(i):
            return core + _NC * j_of(i)

        def u_at(cid):
            return u_hbm.at[pl.ds(cid * _GCHUNK, _GCHUNK)]

        @pl.when(j_of(0) < per_core)
        def _():
            pltpu.async_copy(col_hbm.at[cid_of(0)], idx0, is0)
            pltpu.async_copy(u_at(cid_of(0)), data0, ds0)

        def half(i, b):
            bo = 1 - b

            @pl.when(j_of(i + 1) < per_core)
            def _():
                pltpu.async_copy(col_hbm.at[cid_of(i + 1)], idxs[bo], isems[bo])
                pltpu.async_copy(u_at(cid_of(i + 1)), datas[bo], dsems[bo])

            @pl.when(j_of(i) < per_core)
            def _():
                pltpu.make_async_copy(col_hbm.at[cid_of(i)], idxs[b], isems[b]).wait()
                pltpu.make_async_copy(u_at(cid_of(i)), datas[b], dsems[b]).wait()
                pltpu.sync_copy(datas[b], acc.at[idxs[b]], add=True)

        def body(o, carry):
            half(2 * o, 0)
            half(2 * o + 1, 1)
            return carry

        per_tile = (per_core_max + _NS - 1) // _NS
        lax.fori_loop(0, (per_tile + 1) // 2 + 1, body, 0)
        plsc.subcore_barrier()

        def wblk(i, c):
            k = sid + _NS * i

            @pl.when(k < n_zchunks)
            def _():
                r0 = k * _ZCHUNK
                pltpu.sync_copy(acc.at[pl.ds(r0, _ZCHUNK)], data0.at[pl.ds(0, _ZCHUNK)])
                pltpu.sync_copy(
                    data0.at[pl.ds(0, _ZCHUNK)], s_hbm.at[core, pl.ds(r0, _ZCHUNK)]
                )

            return c

        lax.fori_loop(0, (n_zchunks + _NS - 1) // _NS, wblk, 0)

    return k(u2, col2d)


def _edge_block(
    xg_ref, ea_ref, w1x_ref, w1e_ref, b1_ref, wc0_ref, wc1_ref, wc2_ref, cvec_ref,
    o01_ref, o2_ref,
):
    xb = xg_ref[...].astype(jnp.bfloat16)
    t = jnp.dot(xb, w1x_ref[...], preferred_element_type=jnp.float32)
    t += jnp.dot(
        ea_ref[...].astype(jnp.bfloat16), w1e_ref[...], preferred_element_type=jnp.float32
    )
    t = jnp.maximum(t + b1_ref[...], 0.0).astype(jnp.bfloat16)
    o01_ref[0] = jnp.dot(t, wc0_ref[...], preferred_element_type=jnp.float32)
    o01_ref[1] = jnp.dot(t, wc1_ref[...], preferred_element_type=jnp.float32)
    o2_ref[...] = (
        jnp.dot(t, wc2_ref[...], preferred_element_type=jnp.float32) + cvec_ref[...]
    )


def _edge_stage(xg, ea, W1a_x, W1a_e, b1a, Wc0, Wc1, Wc2p, cvec2):
    n_edges = xg.shape[0]
    grid = (n_edges // N_EDGE_BLK,)
    return pl.pallas_call(
        _edge_block,
        grid=grid,
        in_specs=[
            pl.BlockSpec((N_EDGE_BLK, D_X), lambda i: (i, 0)),
            pl.BlockSpec((N_EDGE_BLK, D_E), lambda i: (i, 0)),
            pl.BlockSpec((D_X, D_H), lambda i: (0, 0)),
            pl.BlockSpec((D_E, D_H), lambda i: (0, 0)),
            pl.BlockSpec((D_H,), lambda i: (0,)),
            pl.BlockSpec((D_H, D_SLAB), lambda i: (0, 0)),
            pl.BlockSpec((D_H, D_SLAB), lambda i: (0, 0)),
            pl.BlockSpec((D_H, D_TAIL), lambda i: (0, 0)),
            pl.BlockSpec((D_TAIL,), lambda i: (0,)),
        ],
        out_specs=[
            pl.BlockSpec((2, N_EDGE_BLK, D_SLAB), lambda i: (0, i, 0)),
            pl.BlockSpec((N_EDGE_BLK, D_TAIL), lambda i: (i, 0)),
        ],
        out_shape=[
            jax.ShapeDtypeStruct((2, n_edges, D_SLAB), jnp.float32),
            jax.ShapeDtypeStruct((n_edges, D_TAIL), jnp.float32),
        ],
    )(xg, ea, W1a_x, W1a_e, b1a, Wc0, Wc1, Wc2p, cvec2)


def _node_block(
    x_ref, sa01_ref, sb01_ref, sa2_ref, sb2_ref, w2x_ref, b2a_ref, bc_ref, w2b_ref,
    b2b_ref, o_ref,
):
    s2 = sa2_ref[0] + sa2_ref[1] + sb2_ref[0] + sb2_ref[1]
    cnt = s2[:, 16:17]
    cntc = jnp.maximum(cnt, 1.0)
    s_u = jnp.concatenate([sa01_ref[...] + sb01_ref[...], s2[:, :16]], axis=1)
    mean_u = s_u / cntc + jnp.where(cnt > 0.0, bc_ref[...][None, :], 0.0)
    h = jnp.dot(x_ref[...], w2x_ref[...], preferred_element_type=jnp.float32)
    h = jnp.maximum(h + mean_u + b2a_ref[...], 0.0)
    o_ref[...] = (
        jnp.dot(h, w2b_ref[...], preferred_element_type=jnp.float32) + b2b_ref[...]
    )


def _node_stage(x, SA01, SB01, SA2, SB2, W2a_x, b2a, bc, W2b, b2b):
    n_nodes = x.shape[0]
    grid = (n_nodes // N_NODE_BLK,)
    return pl.pallas_call(
        _node_block,
        grid=grid,
        in_specs=[
            pl.BlockSpec((N_NODE_BLK, D_X), lambda i: (i, 0)),
            pl.BlockSpec((N_NODE_BLK, 2 * D_SLAB), lambda i: (i, 0)),
            pl.BlockSpec((N_NODE_BLK, 2 * D_SLAB), lambda i: (i, 0)),
            pl.BlockSpec((2, N_NODE_BLK, D_TAIL), lambda i: (0, i, 0)),
            pl.BlockSpec((2, N_NODE_BLK, D_TAIL), lambda i: (0, i, 0)),
            pl.BlockSpec((D_X, D_U), lambda i: (0, 0)),
            pl.BlockSpec((D_U,), lambda i: (0,)),
            pl.BlockSpec((D_U,), lambda i: (0,)),
            pl.BlockSpec((D_U, D_X), lambda i: (0, 0)),
            pl.BlockSpec((D_X,), lambda i: (0,)),
        ],
        out_specs=pl.BlockSpec((N_NODE_BLK, D_X), lambda i: (i, 0)),
        out_shape=jax.ShapeDtypeStruct((n_nodes, D_X), jnp.float32),
    )(x, SA01, SB01, SA2, SB2, W2a_x, b2a, bc, W2b, b2b)


def kernel(x, edge_index, edge_attr, W1a, b1a, W1b, b1b, W2a, b2a, W2b, b2b):
    n_nodes = x.shape[0]
    row = edge_index[0].astype(jnp.int32)
    col = edge_index[1].astype(jnp.int32)

    W1a_x, W1a_e = W1a[:D_X], W1a[D_X:]
    W2a_x, W2a_m = W2a[:D_X], W2a[D_X:]
    Wc = W1b @ W2a_m
    bc = b1b @ W2a_m
    Wc0 = Wc[:, :D_SLAB].astype(jnp.bfloat16)
    Wc1 = Wc[:, D_SLAB : 2 * D_SLAB].astype(jnp.bfloat16)
    Wc2p = jnp.pad(Wc[:, 2 * D_SLAB :], ((0, 0), (0, D_TAIL - 16))).astype(jnp.bfloat16)
    cvec2 = (jnp.arange(D_TAIL) == 16).astype(jnp.float32)
    W1a_x = W1a_x.astype(jnp.bfloat16)
    W1a_e = W1a_e.astype(jnp.bfloat16)

    row2d = row.reshape(-1, _GCHUNK)
    col2d = col.reshape(-1, _GCHUNK)
    nch = row2d.shape[0]
    ha = nch // 2
    e_half = ha * _GCHUNK

    xgA = _sc_gather(x, row2d[:ha])
    xgB = _sc_gather(x, row2d[ha:])
    uA01, uA2 = _edge_stage(
        xgA, edge_attr[:e_half], W1a_x, W1a_e, b1a, Wc0, Wc1, Wc2p, cvec2
    )
    SA01 = _sc_scatter_main(uA01, col2d[:ha], n_nodes)
    SA2 = _sc_scatter_tail(uA2, col2d[:ha], n_nodes)
    uB01, uB2 = _edge_stage(
        xgB, edge_attr[e_half:], W1a_x, W1a_e, b1a, Wc0, Wc1, Wc2p, cvec2
    )
    SB01 = _sc_scatter_main(uB01, col2d[ha:], n_nodes)
    SB2 = _sc_scatter_tail(uB2, col2d[ha:], n_nodes)
    return _node_stage(x, SA01, SB01, SA2, SB2, W2a_x, b2a, bc, W2b, b2b)


# speedup vs baseline: 1.1623x; 1.0347x over previous
import functools

import jax
import jax.numpy as jnp
from jax import lax
from jax.experimental import pallas as pl
from jax.experimental.pallas import tpu as pltpu
from jax.experimental.pallas import tpu_sc as plsc

_NC = 2
_NS = 16
_NW = _NC * _NS
_GCHUNK = 128
_ZCHUNK = 80

N_EDGE_BLK = 1600
N_NODE_BLK = 1000
D_X = 256
D_E = 16
D_H = 544
D_U = 272
D_SLAB = 128
D_TAIL = 128


def _sc_gather(x, idx2d):
    n_chunks = idx2d.shape[0]
    d = x.shape[1]
    per_w = (n_chunks + _NW - 1) // _NW
    mesh = plsc.VectorSubcoreMesh(core_axis_name="c", subcore_axis_name="s")

    @functools.partial(
        pl.kernel,
        mesh=mesh,
        out_type=jax.ShapeDtypeStruct((n_chunks * _GCHUNK, d), jnp.float32),
        scratch_types=[
            pltpu.VMEM((_GCHUNK,), jnp.int32),
            pltpu.VMEM((_GCHUNK,), jnp.int32),
            pltpu.VMEM((_GCHUNK, d), jnp.float32),
            pltpu.VMEM((_GCHUNK, d), jnp.float32),
            pltpu.SemaphoreType.DMA,
            pltpu.SemaphoreType.DMA,
            pltpu.SemaphoreType.DMA,
            pltpu.SemaphoreType.DMA,
            pltpu.SemaphoreType.DMA,
            pltpu.SemaphoreType.DMA,
        ],
    )
    def k(x_hbm, idx_hbm, out_hbm, idx0, idx1, rows0, rows1, is0, is1, gs0, gs1, ws0, ws1):
        wid = lax.axis_index("s") * _NC + lax.axis_index("c")
        idxs = (idx0, idx1)
        rows = (rows0, rows1)
        isems = (is0, is1)
        gsems = (gs0, gs1)
        wsems = (ws0, ws1)

        def cid_of(i):
            return wid + _NW * i

        @pl.when(cid_of(0) < n_chunks)
        def _():
            pltpu.async_copy(idx_hbm.at[cid_of(0)], idx0, is0)

        def half(i, b):
            cid = cid_of(i)
            cidm1 = cid_of(i - 1)
            cidm2 = cid_of(i - 2)
            bo = 1 - b

            @pl.when(jnp.logical_and(i >= 2, cidm2 < n_chunks))
            def _():
                pltpu.make_async_copy(
                    rows[b], out_hbm.at[pl.ds(cidm2 * _GCHUNK, _GCHUNK)], wsems[b]
                ).wait()

            @pl.when(cid < n_chunks)
            def _():
                pltpu.make_async_copy(idx_hbm.at[cid], idxs[b], isems[b]).wait()
                pltpu.async_copy(x_hbm.at[idxs[b]], rows[b], gsems[b])

            @pl.when(jnp.logical_and(i >= 1, cidm1 < n_chunks))
            def _():
                pltpu.make_async_copy(x_hbm.at[idxs[bo]], rows[bo], gsems[bo]).wait()
                pltpu.async_copy(
                    rows[bo], out_hbm.at[pl.ds(cidm1 * _GCHUNK, _GCHUNK)], wsems[bo]
                )

            @pl.when(cid_of(i + 1) < n_chunks)
            def _():
                pltpu.async_copy(idx_hbm.at[cid_of(i + 1)], idxs[bo], isems[bo])

        def body(o, carry):
            half(2 * o, 0)
            half(2 * o + 1, 1)
            return carry

        lax.fori_loop(0, (per_w + 2 + 1) // 2 + 1, body, 0)

    return k(x, idx2d)


def _zero_acc(data_v, acc, sid, n_zchunks, width, zsem):
    def zrow(i, c):
        for j in range(width // 16):
            data_v[i, pl.ds(j * 16, 16)] = jnp.zeros((16,), jnp.float32)
        return c

    lax.fori_loop(0, _ZCHUNK, zrow, 0)

    def zblk(i, c):
        k = sid + _NS * i

        @pl.when(k < n_zchunks)
        def _():
            pltpu.async_copy(
                data_v.at[pl.ds(0, _ZCHUNK)], acc.at[pl.ds(k * _ZCHUNK, _ZCHUNK)], zsem
            )

        return c

    n_it = (n_zchunks + _NS - 1) // _NS
    lax.fori_loop(0, n_it, zblk, 0)

    def zdrain(i, c):
        k = sid + _NS * i

        @pl.when(k < n_zchunks)
        def _():
            pltpu.make_async_copy(
                data_v.at[pl.ds(0, _ZCHUNK)], acc.at[pl.ds(k * _ZCHUNK, _ZCHUNK)], zsem
            ).wait()

        return c

    lax.fori_loop(0, n_it, zdrain, 0)


def _sc_scatter_main(u01, col2d, n_nodes):
    n_chunks = col2d.shape[0]
    n_zchunks = n_nodes // _ZCHUNK
    mesh = plsc.VectorSubcoreMesh(core_axis_name="c", subcore_axis_name="s")

    @functools.partial(
        pl.kernel,
        mesh=mesh,
        out_type=jax.ShapeDtypeStruct((n_nodes, 2 * D_SLAB), jnp.float32),
        scratch_types=[
            pltpu.VMEM((_GCHUNK,), jnp.int32),
            pltpu.VMEM((_GCHUNK,), jnp.int32),
            pltpu.VMEM((_GCHUNK, D_SLAB), jnp.float32),
            pltpu.VMEM((_GCHUNK, D_SLAB), jnp.float32),
            pltpu.VMEM_SHARED((n_nodes, D_SLAB), jnp.float32),
            pltpu.SemaphoreType.DMA,
            pltpu.SemaphoreType.DMA,
            pltpu.SemaphoreType.DMA,
            pltpu.SemaphoreType.DMA,
        ],
    )
    def k(u_hbm, col_hbm, s_hbm, idx0, idx1, data0, data1, acc, is0, is1, ds0, ds1):
        core = lax.axis_index("c")
        sid = lax.axis_index("s")
        idxs = (idx0, idx1)
        datas = (data0, data1)
        isems = (is0, is1)
        dsems = (ds0, ds1)

        _zero_acc(data0, acc, sid, n_zchunks, D_SLAB, is0)
        plsc.subcore_barrier()

        def cid_of(i):
            return sid + _NS * i

        def u_at(cid):
            return u_hbm.at[core, pl.ds(cid * _GCHUNK, _GCHUNK)]

        @pl.when(cid_of(0) < n_chunks)
        def _():
            pltpu.async_copy(col_hbm.at[cid_of(0)], idx0, is0)
            pltpu.async_copy(u_at(cid_of(0)), data0, ds0)

        def half(i, b):
            bo = 1 - b

            @pl.when(cid_of(i + 1) < n_chunks)
            def _():
                pltpu.async_copy(col_hbm.at[cid_of(i + 1)], idxs[bo], isems[bo])
                pltpu.async_copy(u_at(cid_of(i + 1)), datas[bo], dsems[bo])

            @pl.when(cid_of(i) < n_chunks)
            def _():
                pltpu.make_async_copy(col_hbm.at[cid_of(i)], idxs[b], isems[b]).wait()
                pltpu.make_async_copy(u_at(cid_of(i)), datas[b], dsems[b]).wait()
                pltpu.sync_copy(datas[b], acc.at[idxs[b]], add=True)

        def body(o, carry):
            half(2 * o, 0)
            half(2 * o + 1, 1)
            return carry

        per_tile = (n_chunks + _NS - 1) // _NS
        lax.fori_loop(0, (per_tile + 1) // 2 + 1, body, 0)
        plsc.subcore_barrier()

        def s_dst(kk):
            return s_hbm.at[pl.ds(kk * _ZCHUNK, _ZCHUNK), pl.ds(core * D_SLAB, D_SLAB)]

        def whalf(i, b):
            k = sid + _NS * i
            km2 = sid + _NS * (i - 2)

            @pl.when(jnp.logical_and(i >= 2, km2 < n_zchunks))
            def _():
                pltpu.make_async_copy(
                    datas[b].at[pl.ds(0, _ZCHUNK)], s_dst(km2), dsems[b]
                ).wait()

            @pl.when(k < n_zchunks)
            def _():
                pltpu.sync_copy(
                    acc.at[pl.ds(k * _ZCHUNK, _ZCHUNK)], datas[b].at[pl.ds(0, _ZCHUNK)]
                )
                pltpu.async_copy(datas[b].at[pl.ds(0, _ZCHUNK)], s_dst(k), dsems[b])

            return 0

        def wbody(o, c):
            whalf(2 * o, 0)
            whalf(2 * o + 1, 1)
            return c

        n_wit = (n_zchunks + _NS - 1) // _NS
        lax.fori_loop(0, (n_wit + 2 + 1) // 2 + 1, wbody, 0)

    return k(u01, col2d)


def _sc_scatter_tail(u2, col2d, n_nodes):
    n_chunks = col2d.shape[0]
    per_core_max = (n_chunks + _NC - 1) // _NC
    n_zchunks = n_nodes // _ZCHUNK
    mesh = plsc.VectorSubcoreMesh(core_axis_name="c", subcore_axis_name="s")

    @functools.partial(
        pl.kernel,
        mesh=mesh,
        out_type=jax.ShapeDtypeStruct((_NC, n_nodes, D_TAIL), jnp.float32),
        scratch_types=[
            pltpu.VMEM((_GCHUNK,), jnp.int32),
            pltpu.VMEM((_GCHUNK,), jnp.int32),
            pltpu.VMEM((_GCHUNK, D_TAIL), jnp.float32),
            pltpu.VMEM((_GCHUNK, D_TAIL), jnp.float32),
            pltpu.VMEM_SHARED((n_nodes, D_TAIL), jnp.float32),
            pltpu.SemaphoreType.DMA,
            pltpu.SemaphoreType.DMA,
            pltpu.SemaphoreType.DMA,
            pltpu.SemaphoreType.DMA,
        ],
    )
    def k(u_hbm, col_hbm, s_hbm, idx0, idx1, data0, data1, acc, is0, is1, ds0, ds1):
        core = lax.axis_index("c")
        sid = lax.axis_index("s")
        idxs = (idx0, idx1)
        datas = (data0, data1)
        isems = (is0, is1)
        dsems = (ds0, ds1)

        _zero_acc(data0, acc, sid, n_zchunks, D_TAIL, is0)
        plsc.subcore_barrier()

        def j_of(i):
            return sid + _NS * i

        per_core = (n_chunks - core + _NC - 1) // _NC

        def cid_of(i):
            return core + _NC * j_of(i)

        def u_at(cid):
            return u_hbm.at[pl.ds(cid * _GCHUNK, _GCHUNK)]

        @pl.when(j_of(0) < per_core)
        def _():
            pltpu.async_copy(col_hbm.at[cid_of(0)], idx0, is0)
            pltpu.async_copy(u_at(cid_of(0)), data0, ds0)

        def half(i, b):
            bo = 1 - b

            @pl.when(j_of(i + 1) < per_core)
            def _():
                pltpu.async_copy(col_hbm.at[cid_of(i + 1)], idxs[bo], isems[bo])
                pltpu.async_copy(u_at(cid_of(i + 1)), datas[bo], dsems[bo])

            @pl.when(j_of(i) < per_core)
            def _():
                pltpu.make_async_copy(col_hbm.at[cid_of(i)], idxs[b], isems[b]).wait()
                pltpu.make_async_copy(u_at(cid_of(i)), datas[b], dsems[b]).wait()
                pltpu.sync_copy(datas[b], acc.at[idxs[b]], add=True)

        def body(o, carry):
            half(2 * o, 0)
            half(2 * o + 1, 1)
            return carry

        per_tile = (per_core_max + _NS - 1) // _NS
        lax.fori_loop(0, (per_tile + 1) // 2 + 1, body, 0)
        plsc.subcore_barrier()

        def s_dst(kk):
            return s_hbm.at[core, pl.ds(kk * _ZCHUNK, _ZCHUNK)]

        def whalf(i, b):
            k = sid + _NS * i
            km2 = sid + _NS * (i - 2)

            @pl.when(jnp.logical_and(i >= 2, km2 < n_zchunks))
            def _():
                pltpu.make_async_copy(
                    datas[b].at[pl.ds(0, _ZCHUNK)], s_dst(km2), dsems[b]
                ).wait()

            @pl.when(k < n_zchunks)
            def _():
                pltpu.sync_copy(
                    acc.at[pl.ds(k * _ZCHUNK, _ZCHUNK)], datas[b].at[pl.ds(0, _ZCHUNK)]
                )
                pltpu.async_copy(datas[b].at[pl.ds(0, _ZCHUNK)], s_dst(k), dsems[b])

            return 0

        def wbody(o, c):
            whalf(2 * o, 0)
            whalf(2 * o + 1, 1)
            return c

        n_wit = (n_zchunks + _NS - 1) // _NS
        lax.fori_loop(0, (n_wit + 2 + 1) // 2 + 1, wbody, 0)

    return k(u2, col2d)


def _edge_block(
    xg_ref, ea_ref, w1x_ref, w1e_ref, b1_ref, wc0_ref, wc1_ref, wc2_ref, cvec_ref,
    o01_ref, o2_ref,
):
    xb = xg_ref[...].astype(jnp.bfloat16)
    t = jnp.dot(xb, w1x_ref[...], preferred_element_type=jnp.float32)
    t += jnp.dot(
        ea_ref[...].astype(jnp.bfloat16), w1e_ref[...], preferred_element_type=jnp.float32
    )
    t = jnp.maximum(t + b1_ref[...], 0.0).astype(jnp.bfloat16)
    o01_ref[0] = jnp.dot(t, wc0_ref[...], preferred_element_type=jnp.float32)
    o01_ref[1] = jnp.dot(t, wc1_ref[...], preferred_element_type=jnp.float32)
    o2_ref[...] = (
        jnp.dot(t, wc2_ref[...], preferred_element_type=jnp.float32) + cvec_ref[...]
    )


def _edge_stage(xg, ea, ea_off_blocks, W1a_x, W1a_e, b1a, Wc0, Wc1, Wc2p, cvec2):
    n_edges = xg.shape[0]
    grid = (n_edges // N_EDGE_BLK,)
    return pl.pallas_call(
        _edge_block,
        grid=grid,
        in_specs=[
            pl.BlockSpec((N_EDGE_BLK, D_X), lambda i: (i, 0)),
            pl.BlockSpec((N_EDGE_BLK, D_E), lambda i: (i + ea_off_blocks, 0)),
            pl.BlockSpec((D_X, D_H), lambda i: (0, 0)),
            pl.BlockSpec((D_E, D_H), lambda i: (0, 0)),
            pl.BlockSpec((D_H,), lambda i: (0,)),
            pl.BlockSpec((D_H, D_SLAB), lambda i: (0, 0)),
            pl.BlockSpec((D_H, D_SLAB), lambda i: (0, 0)),
            pl.BlockSpec((D_H, D_TAIL), lambda i: (0, 0)),
            pl.BlockSpec((D_TAIL,), lambda i: (0,)),
        ],
        out_specs=[
            pl.BlockSpec((2, N_EDGE_BLK, D_SLAB), lambda i: (0, i, 0)),
            pl.BlockSpec((N_EDGE_BLK, D_TAIL), lambda i: (i, 0)),
        ],
        out_shape=[
            jax.ShapeDtypeStruct((2, n_edges, D_SLAB), jnp.float32),
            jax.ShapeDtypeStruct((n_edges, D_TAIL), jnp.float32),
        ],
    )(xg, ea, W1a_x, W1a_e, b1a, Wc0, Wc1, Wc2p, cvec2)


def _node_block(
    x_ref, sa01_ref, sb01_ref, sa2_ref, sb2_ref, w2x_ref, b2a_ref, bc_ref, w2b_ref,
    b2b_ref, o_ref,
):
    s2 = sa2_ref[0] + sa2_ref[1] + sb2_ref[0] + sb2_ref[1]
    cnt = s2[:, 16:17]
    cntc = jnp.maximum(cnt, 1.0)
    s_u = jnp.concatenate([sa01_ref[...] + sb01_ref[...], s2[:, :16]], axis=1)
    mean_u = s_u / cntc + jnp.where(cnt > 0.0, bc_ref[...][None, :], 0.0)
    h = jnp.dot(x_ref[...], w2x_ref[...], preferred_element_type=jnp.float32)
    h = jnp.maximum(h + mean_u + b2a_ref[...], 0.0)
    o_ref[...] = (
        jnp.dot(h, w2b_ref[...], preferred_element_type=jnp.float32) + b2b_ref[...]
    )


def _node_stage(x, SA01, SB01, SA2, SB2, W2a_x, b2a, bc, W2b, b2b):
    n_nodes = x.shape[0]
    grid = (n_nodes // N_NODE_BLK,)
    return pl.pallas_call(
        _node_block,
        grid=grid,
        in_specs=[
            pl.BlockSpec((N_NODE_BLK, D_X), lambda i: (i, 0)),
            pl.BlockSpec((N_NODE_BLK, 2 * D_SLAB), lambda i: (i, 0)),
            pl.BlockSpec((N_NODE_BLK, 2 * D_SLAB), lambda i: (i, 0)),
            pl.BlockSpec((2, N_NODE_BLK, D_TAIL), lambda i: (0, i, 0)),
            pl.BlockSpec((2, N_NODE_BLK, D_TAIL), lambda i: (0, i, 0)),
            pl.BlockSpec((D_X, D_U), lambda i: (0, 0)),
            pl.BlockSpec((D_U,), lambda i: (0,)),
            pl.BlockSpec((D_U,), lambda i: (0,)),
            pl.BlockSpec((D_U, D_X), lambda i: (0, 0)),
            pl.BlockSpec((D_X,), lambda i: (0,)),
        ],
        out_specs=pl.BlockSpec((N_NODE_BLK, D_X), lambda i: (i, 0)),
        out_shape=jax.ShapeDtypeStruct((n_nodes, D_X), jnp.float32),
    )(x, SA01, SB01, SA2, SB2, W2a_x, b2a, bc, W2b, b2b)


def kernel(x, edge_index, edge_attr, W1a, b1a, W1b, b1b, W2a, b2a, W2b, b2b):
    n_nodes = x.shape[0]
    row = edge_index[0].astype(jnp.int32)
    col = edge_index[1].astype(jnp.int32)

    W1a_x, W1a_e = W1a[:D_X], W1a[D_X:]
    W2a_x, W2a_m = W2a[:D_X], W2a[D_X:]
    Wc = W1b @ W2a_m
    bc = b1b @ W2a_m
    Wc0 = Wc[:, :D_SLAB].astype(jnp.bfloat16)
    Wc1 = Wc[:, D_SLAB : 2 * D_SLAB].astype(jnp.bfloat16)
    Wc2p = jnp.pad(Wc[:, 2 * D_SLAB :], ((0, 0), (0, D_TAIL - 16))).astype(jnp.bfloat16)
    cvec2 = (jnp.arange(D_TAIL) == 16).astype(jnp.float32)
    W1a_x = W1a_x.astype(jnp.bfloat16)
    W1a_e = W1a_e.astype(jnp.bfloat16)

    row2d = row.reshape(-1, _GCHUNK)
    col2d = col.reshape(-1, _GCHUNK)
    nch = row2d.shape[0]
    ha = nch // 2
    e_half = ha * _GCHUNK

    xgA = _sc_gather(x, row2d[:ha])
    xgB = _sc_gather(x, row2d[ha:])
    uA01, uA2 = _edge_stage(xgA, edge_attr, 0, W1a_x, W1a_e, b1a, Wc0, Wc1, Wc2p, cvec2)
    SA01 = _sc_scatter_main(uA01, col2d[:ha], n_nodes)
    SA2 = _sc_scatter_tail(uA2, col2d[:ha], n_nodes)
    uB01, uB2 = _edge_stage(
        xgB, edge_attr, e_half // N_EDGE_BLK, W1a_x, W1a_e, b1a, Wc0, Wc1, Wc2p, cvec2
    )
    SB01 = _sc_scatter_main(uB01, col2d[ha:], n_nodes)
    SB2 = _sc_scatter_tail(uB2, col2d[ha:], n_nodes)
    return _node_stage(x, SA01, SB01, SA2, SB2, W2a_x, b2a, bc, W2b, b2b)
